# baseline (speedup 1.0000x reference)
import functools

import jax
import jax.numpy as jnp
from jax import lax
from jax.experimental import pallas as pl
from jax.experimental.pallas import tpu as pltpu
from jax.experimental.pallas import tpu_sc as plsc

_TV = 50265
_V = 52267
_B = 4
_S = 250
_R = _B * _S
_TOTAL = _R * _V
_NW = 32
_RPW = 32
_CHUNK = 52288
_BUF = 52304

_TEXT_BLOCKS = 3141
_BOUND_COL = 50256
_BOX_LO = 3142
_BOX_HI = 3266
_TAIL_COL = 52256


def _sc_body(logits, out, buf, res):
    info = plsc.get_sparse_core_info()
    nc = info.num_cores
    wid = lax.axis_index("s") * nc + lax.axis_index("c")
    lane = lax.broadcasted_iota(jnp.int32, (16,), 0)
    neg = jnp.full((16,), -jnp.inf, jnp.float32)

    def row_body(i, carry):
        r = jnp.minimum(wid * _RPW + i, _R - 1)
        start = r * _V
        start_al = jnp.minimum((start // 8) * 8, _TOTAL - _CHUNK)
        d = start - start_al
        pltpu.sync_copy(logits.at[pl.ds(start_al, _CHUNK)],
                        buf.at[pl.ds(0, _CHUNK)])

        def maxloop(lo, hi, acc):
            def body(k, a):
                return jnp.maximum(a, buf[pl.ds(d + k * 16, 16)])
            return lax.fori_loop(lo, hi, body, acc)

        acc_t = maxloop(0, _TEXT_BLOCKS, neg)
        bvec = buf[pl.ds(d + _BOUND_COL, 16)]
        acc_t = jnp.maximum(acc_t, jnp.where(lane <= 8, bvec, neg))
        m_text = jnp.max(acc_t)
        v_open = jnp.max(jnp.where(lane == 9, bvec, neg))
        v_close = jnp.max(jnp.where(lane == 10, bvec, neg))
        acc_b = jnp.where(lane >= 11, bvec, neg)
        acc_b = maxloop(_BOX_LO, _BOX_HI, acc_b)
        tvec = buf[pl.ds(d + _TAIL_COL, 16)]
        acc_b = jnp.maximum(acc_b, jnp.where(lane <= 10, tvec, neg))
        m_box = jnp.max(acc_b)

        packed = jnp.where(lane == 0, m_text,
                 jnp.where(lane == 1, v_open,
                 jnp.where(lane == 2, v_close, m_box)))
        res[pl.ds(i * 16, 16)] = packed
        return carry

    lax.fori_loop(0, _RPW, row_body, 0)
    pltpu.sync_copy(res, out.at[pl.ds(wid * (_RPW * 16), _RPW * 16)])


@functools.partial(jax.jit, static_argnums=())
def _segment_maxes(logits_flat):
    call = pl.kernel(
        _sc_body,
        out_type=jax.ShapeDtypeStruct((_NW * _RPW * 16,), jnp.float32),
        mesh=plsc.VectorSubcoreMesh(core_axis_name="c", subcore_axis_name="s"),
        compiler_params=pltpu.CompilerParams(needs_layout_passes=False),
        scratch_types=[
            pltpu.VMEM((_BUF,), jnp.float32),
            pltpu.VMEM((_RPW * 16,), jnp.float32),
        ],
    )
    return call(logits_flat)


def _tc_body(mt_ref, mo_ref, mc_ref, mb_ref, bt_ref, ts_ref,
             scores_ref, boxes_ref):
    f32 = jnp.float32
    mt = mt_ref[...]
    mo = mo_ref[...]
    mc = mc_ref[...]
    mb = mb_ref[...]

    is_text = mt >= jnp.maximum(jnp.maximum(mo, mc), mb)
    is_open = jnp.logical_not(is_text) & (mo >= jnp.maximum(mc, mb))
    is_close = jnp.logical_not(is_text | is_open) & (mc >= mb)
    is_box = jnp.logical_not(is_text | is_open | is_close)

    ft = is_text.astype(f32)
    fo = is_open.astype(f32)
    fc = is_close.astype(f32)
    fb = is_box.astype(f32)

    jj = lax.broadcasted_iota(jnp.int32, (_S, _S), 0)
    ii = lax.broadcasted_iota(jnp.int32, (_S, _S), 1)
    U = (jj < ii).astype(f32)

    def excl(x):
        return jnp.dot(x, U, preferred_element_type=f32)

    in_bbox = excl(fo - fc)
    box_cnt = excl(fb)
    str_idx = excl(ft)
    total_box = jnp.sum(fb, axis=1, keepdims=True)
    num_rows = jnp.floor((total_box + 3.0) / 4.0)
    box_idx = jnp.floor(box_cnt / 4.0)
    valid = is_text & (in_bbox != 0.0) & (box_idx < num_rows)
    del str_idx

    bi = jnp.where(valid, box_idx, f32(1e6))
    rvec = lax.broadcasted_iota(jnp.int32, (1, 1, 100), 2).astype(f32)
    k = jnp.sum((bi[:, :, None] == rvec).astype(f32), axis=1)

    e1 = jnp.exp(f32(1.0)) - f32(1.0)
    scores_ref[...] = f32(1.0) - f32(1.0) / (f32(256.0) + k * e1)

    ts = ts_ref[...].astype(f32)
    h = ts[:, 0:1]
    w = ts[:, 1:2]
    bt = bt_ref[...]
    xc = bt[:, 0, :]
    yc = bt[:, 1, :]
    bw = bt[:, 2, :]
    bh = bt[:, 3, :]
    boxes_ref[:, 0, :] = (xc - 0.5 * bw) * w
    boxes_ref[:, 1, :] = (yc - 0.5 * bh) * h
    boxes_ref[:, 2, :] = (xc + 0.5 * bw) * w
    boxes_ref[:, 3, :] = (yc + 0.5 * bh) * h


def _post(mt, mo, mc, mb, bt, ts):
    return pl.pallas_call(
        _tc_body,
        out_shape=[
            jax.ShapeDtypeStruct((_B, 100), jnp.float32),
            jax.ShapeDtypeStruct((_B, 4, 100), jnp.float32),
        ],
    )(mt, mo, mc, mb, bt, ts)


def kernel(pred_logits, pred_boxes, target_sizes):
    logits_flat = pred_logits.reshape(-1)
    maxes = _segment_maxes(logits_flat)
    m = maxes.reshape(_NW * _RPW, 16)[:_R, :4].reshape(_B, _S, 4)
    bt = jnp.transpose(pred_boxes, (0, 2, 1))
    scores, boxes_t = _post(m[..., 0], m[..., 1], m[..., 2], m[..., 3],
                            bt, target_sizes)
    boxes = jnp.transpose(boxes_t, (0, 2, 1))
    labels = jnp.ones((_B, 100), jnp.int32)
    return scores, labels, boxes


# speedup vs baseline: 1.1717x; 1.1717x over previous
import functools

import jax
import jax.numpy as jnp
from jax import lax
from jax.experimental import pallas as pl
from jax.experimental.pallas import tpu as pltpu
from jax.experimental.pallas import tpu_sc as plsc

_TV = 50265
_V = 52267
_B = 4
_S = 250
_R = _B * _S
_TOTAL = _R * _V
_NW = 32
_RPW = 32
_CHUNK = 52288
_BUF = 52304

_TEXT_BLOCKS = 3141
_BOUND_COL = 50256
_BOX_LO = 3142
_BOX_HI = 3266
_TAIL_COL = 52256


def _row_reduce(buf, d, lane, neg, res, i):
    def maxloop8(col0, iters, accs):
        def body(k, a):
            base = d + col0 + k * 128
            a0 = jnp.maximum(a[0], buf[pl.ds(base, 16)])
            a1 = jnp.maximum(a[1], buf[pl.ds(base + 16, 16)])
            a2 = jnp.maximum(a[2], buf[pl.ds(base + 32, 16)])
            a3 = jnp.maximum(a[3], buf[pl.ds(base + 48, 16)])
            a0 = jnp.maximum(a0, buf[pl.ds(base + 64, 16)])
            a1 = jnp.maximum(a1, buf[pl.ds(base + 80, 16)])
            a2 = jnp.maximum(a2, buf[pl.ds(base + 96, 16)])
            a3 = jnp.maximum(a3, buf[pl.ds(base + 112, 16)])
            return (a0, a1, a2, a3)
        return lax.fori_loop(0, iters, body, accs)

    def tailblocks(col0, n, acc):
        for t in range(n):
            acc = jnp.maximum(acc, buf[pl.ds(d + col0 + t * 16, 16)])
        return acc

    accs = maxloop8(0, 392, (neg, neg, neg, neg))
    acc_t = jnp.maximum(jnp.maximum(accs[0], accs[1]),
                        jnp.maximum(accs[2], accs[3]))
    acc_t = tailblocks(392 * 128, 5, acc_t)
    bvec = buf[pl.ds(d + _BOUND_COL, 16)]
    acc_t = jnp.maximum(acc_t, jnp.where(lane <= 8, bvec, neg))
    m_text = jnp.max(acc_t)
    v_open = jnp.max(jnp.where(lane == 9, bvec, neg))
    v_close = jnp.max(jnp.where(lane == 10, bvec, neg))
    accs = maxloop8(_BOUND_COL + 16, 15, (neg, neg, neg, neg))
    acc_b = jnp.maximum(jnp.maximum(accs[0], accs[1]),
                        jnp.maximum(accs[2], accs[3]))
    acc_b = tailblocks(_BOUND_COL + 16 + 15 * 128, 4, acc_b)
    acc_b = jnp.maximum(acc_b, jnp.where(lane >= 11, bvec, neg))
    tvec = buf[pl.ds(d + _TAIL_COL, 16)]
    acc_b = jnp.maximum(acc_b, jnp.where(lane <= 10, tvec, neg))
    m_box = jnp.max(acc_b)

    packed = jnp.where(lane == 0, m_text,
             jnp.where(lane == 1, v_open,
             jnp.where(lane == 2, v_close, m_box)))
    res[pl.ds(i * 16, 16)] = packed


def _sc_body(logits, out, buf0, buf1, res, sem0, sem1):
    info = plsc.get_sparse_core_info()
    nc = info.num_cores
    wid = lax.axis_index("s") * nc + lax.axis_index("c")
    lane = lax.broadcasted_iota(jnp.int32, (16,), 0)
    neg = jnp.full((16,), -jnp.inf, jnp.float32)

    def row_window(i):
        r = jnp.minimum(wid * _RPW + i, _R - 1)
        start = r * _V
        start_al = jnp.minimum((start // 8) * 8, _TOTAL - _CHUNK)
        return start_al, start - start_al

    def start_dma(i, buf, sem):
        start_al, _ = row_window(i)
        pltpu.async_copy(logits.at[pl.ds(start_al, _CHUNK)],
                         buf.at[pl.ds(0, _CHUNK)], sem)

    def wait_dma(buf, sem):
        pltpu.make_async_copy(logits.at[pl.ds(0, _CHUNK)],
                              buf.at[pl.ds(0, _CHUNK)], sem).wait()

    start_dma(0, buf0, sem0)

    def pair_body(j, carry):
        i0 = j * 2
        start_dma(i0 + 1, buf1, sem1)
        wait_dma(buf0, sem0)
        _, d0 = row_window(i0)
        _row_reduce(buf0, d0, lane, neg, res, i0)

        @pl.when(j < _RPW // 2 - 1)
        def _():
            start_dma(i0 + 2, buf0, sem0)

        wait_dma(buf1, sem1)
        _, d1 = row_window(i0 + 1)
        _row_reduce(buf1, d1, lane, neg, res, i0 + 1)
        return carry

    lax.fori_loop(0, _RPW // 2, pair_body, 0)
    pltpu.sync_copy(res, out.at[pl.ds(wid * (_RPW * 16), _RPW * 16)])


@functools.partial(jax.jit, static_argnums=())
def _segment_maxes(logits_flat):
    call = pl.kernel(
        _sc_body,
        out_type=jax.ShapeDtypeStruct((_NW * _RPW * 16,), jnp.float32),
        mesh=plsc.VectorSubcoreMesh(core_axis_name="c", subcore_axis_name="s"),
        compiler_params=pltpu.CompilerParams(needs_layout_passes=False),
        scratch_types=[
            pltpu.VMEM((_BUF,), jnp.float32),
            pltpu.VMEM((_BUF,), jnp.float32),
            pltpu.VMEM((_RPW * 16,), jnp.float32),
            pltpu.SemaphoreType.DMA,
            pltpu.SemaphoreType.DMA,
        ],
    )
    return call(logits_flat)


def _tc_body(mt_ref, mo_ref, mc_ref, mb_ref, bt_ref, ts_ref,
             scores_ref, boxes_ref):
    f32 = jnp.float32
    mt = mt_ref[...]
    mo = mo_ref[...]
    mc = mc_ref[...]
    mb = mb_ref[...]

    is_text = mt >= jnp.maximum(jnp.maximum(mo, mc), mb)
    is_open = jnp.logical_not(is_text) & (mo >= jnp.maximum(mc, mb))
    is_close = jnp.logical_not(is_text | is_open) & (mc >= mb)
    is_box = jnp.logical_not(is_text | is_open | is_close)

    ft = is_text.astype(f32)
    fo = is_open.astype(f32)
    fc = is_close.astype(f32)
    fb = is_box.astype(f32)

    jj = lax.broadcasted_iota(jnp.int32, (_S, _S), 0)
    ii = lax.broadcasted_iota(jnp.int32, (_S, _S), 1)
    U = (jj < ii).astype(f32)

    def excl(x):
        return jnp.dot(x, U, preferred_element_type=f32)

    in_bbox = excl(fo - fc)
    box_cnt = excl(fb)
    str_idx = excl(ft)
    total_box = jnp.sum(fb, axis=1, keepdims=True)
    num_rows = jnp.floor((total_box + 3.0) / 4.0)
    box_idx = jnp.floor(box_cnt / 4.0)
    valid = is_text & (in_bbox != 0.0) & (box_idx < num_rows)
    del str_idx

    bi = jnp.where(valid, box_idx, f32(1e6))
    rvec = lax.broadcasted_iota(jnp.int32, (1, 1, 100), 2).astype(f32)
    k = jnp.sum((bi[:, :, None] == rvec).astype(f32), axis=1)

    e1 = jnp.exp(f32(1.0)) - f32(1.0)
    scores_ref[...] = f32(1.0) - f32(1.0) / (f32(256.0) + k * e1)

    ts = ts_ref[...].astype(f32)
    h = ts[:, 0:1]
    w = ts[:, 1:2]
    bt = bt_ref[...]
    xc = bt[:, 0, :]
    yc = bt[:, 1, :]
    bw = bt[:, 2, :]
    bh = bt[:, 3, :]
    boxes_ref[:, 0, :] = (xc - 0.5 * bw) * w
    boxes_ref[:, 1, :] = (yc - 0.5 * bh) * h
    boxes_ref[:, 2, :] = (xc + 0.5 * bw) * w
    boxes_ref[:, 3, :] = (yc + 0.5 * bh) * h


def _post(mt, mo, mc, mb, bt, ts):
    return pl.pallas_call(
        _tc_body,
        out_shape=[
            jax.ShapeDtypeStruct((_B, 100), jnp.float32),
            jax.ShapeDtypeStruct((_B, 4, 100), jnp.float32),
        ],
    )(mt, mo, mc, mb, bt, ts)


def kernel(pred_logits, pred_boxes, target_sizes):
    logits_flat = pred_logits.reshape(-1)
    maxes = _segment_maxes(logits_flat)
    m = maxes.reshape(_NW * _RPW, 16)[:_R, :4].reshape(_B, _S, 4)
    bt = jnp.transpose(pred_boxes, (0, 2, 1))
    scores, boxes_t = _post(m[..., 0], m[..., 1], m[..., 2], m[..., 3],
                            bt, target_sizes)
    boxes = jnp.transpose(boxes_t, (0, 2, 1))
    labels = jnp.ones((_B, 100), jnp.int32)
    return scores, labels, boxes
